# Initial kernel scaffold; baseline (speedup 1.0000x reference)
import jax
import jax.numpy as jnp
from jax.experimental import pallas as pl


def kernel(batchX, W):
    raise NotImplementedError("write your pallas kernel here")


# speedup vs baseline: 2.6425x; 2.6425x over previous
import functools

import jax
import jax.numpy as jnp
from jax import lax
from jax.experimental import pallas as pl
from jax.experimental.pallas import tpu as pltpu
from jax.experimental.pallas import tpu_sc as plsc


def _bitonic(vs):
    if len(vs) == 1:
        v = vs[0]
        return [None if v is None else lax.sort(v)]
    h = len(vs) // 2
    lo, hi = [], []
    for a, b in zip(vs[:h], vs[h:]):
        if a is None and b is None:
            lo.append(None)
            hi.append(None)
        elif a is None:
            lo.append(b)
            hi.append(None)
        elif b is None:
            lo.append(a)
            hi.append(None)
        else:
            lo.append(jnp.minimum(a, b))
            hi.append(jnp.maximum(a, b))
    return _bitonic(lo) + _bitonic(hi)


def _merge(x, y):
    rev_y = [None if v is None else lax.rev(v, (0,)) for v in reversed(y)]
    return _bitonic(x + rev_y)


@functools.lru_cache(maxsize=None)
def _build_sc_row_sort(B, F):
    info = plsc.get_sparse_core_info()
    NC, NS, L = info.num_cores, info.num_subcores, info.num_lanes
    NW = NC * NS
    assert B % NW == 0
    rows_w = B // NW
    chunk = rows_w * F
    nreg = -(-F // L)
    rem = F - (nreg - 1) * L
    mesh = plsc.VectorSubcoreMesh(core_axis_name="c", subcore_axis_name="s")

    @functools.partial(
        pl.kernel,
        mesh=mesh,
        compiler_params=pltpu.CompilerParams(needs_layout_passes=False),
        out_type=jax.ShapeDtypeStruct((B * F,), jnp.float32),
        scratch_types=[
            pltpu.VMEM((chunk + L,), jnp.float32),
            pltpu.VMEM((chunk + L,), jnp.float32),
        ],
    )
    def k(x_hbm, out_hbm, xin, xout):
        wid = lax.axis_index("s") * NC + lax.axis_index("c")
        base = wid * chunk
        pltpu.sync_copy(x_hbm.at[pl.ds(base, chunk)], xin.at[pl.ds(0, chunk)])
        lane = lax.iota(jnp.int32, L)

        def body(i, carry):
            b = i * F
            regs = [xin[pl.ds(b + j * L, L)] for j in range(nreg - 1)]
            tail = xin[pl.ds(b + (nreg - 1) * L, L)]
            regs.append(jnp.where(lane < rem, tail, jnp.inf))
            s = [lax.sort(v) for v in regs]
            a = _merge([s[0]], [s[1]])
            c = _merge([s[2]], [s[3]])
            d = _merge([s[4]], [s[5]])
            e = _merge(a, c)
            f = _merge(d, [s[6], None])
            g = _merge(e, f)
            for j, v in enumerate(g):
                if v is not None:
                    xout[pl.ds(b + j * L, L)] = v
            return carry

        lax.fori_loop(0, rows_w, body, 0)
        pltpu.sync_copy(xout.at[pl.ds(0, chunk)], out_hbm.at[pl.ds(base, chunk)])

    return k


def kernel(batchX, W):
    B, F = batchX.shape
    out_flat = _build_sc_row_sort(B, F)(batchX.reshape(-1))
    return out_flat.reshape(B, 1, 10, 10)


# speedup vs baseline: 4.8291x; 1.8275x over previous
import functools

import jax
import jax.numpy as jnp
from jax import lax
from jax.experimental import pallas as pl
from jax.experimental.pallas import tpu as pltpu
from jax.experimental.pallas import tpu_sc as plsc


def _bitonic(vs):
    if len(vs) == 1:
        v = vs[0]
        return [None if v is None else lax.sort(v)]
    h = len(vs) // 2
    lo, hi = [], []
    for a, b in zip(vs[:h], vs[h:]):
        if a is None and b is None:
            lo.append(None)
            hi.append(None)
        elif a is None:
            lo.append(b)
            hi.append(None)
        elif b is None:
            lo.append(a)
            hi.append(None)
        else:
            lo.append(jnp.minimum(a, b))
            hi.append(jnp.maximum(a, b))
    return _bitonic(lo) + _bitonic(hi)


def _merge(x, y):
    rev_y = [None if v is None else lax.rev(v, (0,)) for v in reversed(y)]
    return _bitonic(x + rev_y)


@functools.lru_cache(maxsize=None)
def _build_sc_row_sort(B, F):
    info = plsc.get_sparse_core_info()
    NC, NS, L = info.num_cores, info.num_subcores, info.num_lanes
    NW = NC * NS
    assert B % NW == 0
    rows_w = B // NW
    nfull = F // L
    rem = F - nfull * L
    mesh = plsc.VectorSubcoreMesh(core_axis_name="c", subcore_axis_name="s")

    @functools.partial(
        pl.kernel,
        mesh=mesh,
        compiler_params=pltpu.CompilerParams(needs_layout_passes=False),
        out_type=jax.ShapeDtypeStruct((B, F), jnp.float32),
        scratch_types=[
            pltpu.VMEM((rows_w, F), jnp.float32),
            pltpu.VMEM((8 * 2 * L,), jnp.float32),
        ],
    )
    def k(x_hbm, out_hbm, xio, tailbuf):
        wid = lax.axis_index("s") * NC + lax.axis_index("c")
        r0 = wid * rows_w
        pltpu.sync_copy(x_hbm.at[pl.ds(r0, rows_w)], xio)
        lane = lax.iota(jnp.int32, L)

        def body(r, carry):
            regs = [xio[r, pl.ds(j * L, L)] for j in range(nfull)]
            tail = xio[r, pl.ds(F - L, L)]
            regs.append(jnp.where(lane >= L - rem, tail, jnp.inf))
            s = [lax.sort(v) for v in regs]
            a = _merge([s[0]], [s[1]])
            c = _merge([s[2]], [s[3]])
            d = _merge([s[4]], [s[5]])
            e = _merge(a, c)
            f = _merge(d, [s[6], None])
            g = _merge(e, f)
            for j in range(nfull - 1):
                xio[r, pl.ds(j * L, L)] = g[j]
            xio[r, pl.ds((nfull - 1) * L, L)] = g[nfull - 1]
            off = (r & 7) * 2 * L
            tailbuf[pl.ds(off, L)] = g[nfull - 1]
            tailbuf[pl.ds(off + L, L)] = g[nfull]
            xio[r, pl.ds(F - L, L)] = tailbuf[pl.ds(off + rem, L)]
            return carry

        lax.fori_loop(0, rows_w, body, 0)
        pltpu.sync_copy(xio, out_hbm.at[pl.ds(r0, rows_w)])

    return k


def kernel(batchX, W):
    B, F = batchX.shape
    out = _build_sc_row_sort(B, F)(batchX)
    return out.reshape(B, 1, 10, 10)
